# baseline (speedup 1.0000x reference)
import functools

import jax
import jax.numpy as jnp
from jax import lax
from jax.experimental import pallas as pl
from jax.experimental.pallas import tpu as pltpu
from jax.experimental.pallas import tpu_sc as plsc

N_NODES = 10000
N_EDGES = 320000
D_IN = 128
D_EDGE = 16
D_NODE = 18
H_FEA = 16
G_POOL = 64

DP = 32
CNT = D_NODE

NC = 2
NS = 16
NW = NC * NS
EPW = 10240
CH = 320
NCHUNK = EPW // CH
CH4 = CH // 4
NP = 10240
RPS = NP // NS


def _node_tables_body(x_ref, wembT_ref, bemb_ref, w12T_ref, a_ref, b_ref):
    h = (jnp.dot(x_ref[...], wembT_ref[...],
                 preferred_element_type=jnp.float32) + bemb_ref[...])
    ab = jnp.dot(h, w12T_ref[...], preferred_element_type=jnp.float32)
    a_ref[...] = ab[:, :DP]
    b_ref[...] = ab[:, DP:]


def _node_tables(x, wembT, bemb, w12T):
    return pl.pallas_call(
        _node_tables_body,
        out_shape=(
            jax.ShapeDtypeStruct((NP, DP), jnp.float32),
            jax.ShapeDtypeStruct((NP, DP), jnp.float32),
        ),
    )(x, wembT, bemb, w12T)


E4 = N_EDGES // 4
_EBLK4 = 5000
_NBLK = E4 // _EBLK4


def _edge_table_body(e0, e1, e2, e3, w_ref, bias_ref, c_ref):
    parts = []
    for ea_ref in (e0, e1, e2, e3):
        parts.append(
            jnp.dot(ea_ref[...], w_ref[...], preferred_element_type=jnp.float32)
            + bias_ref[...]
        )
    c_ref[...] = jnp.concatenate(parts, axis=1)


def _edge_table(edge_attr, w3p, cbias):
    quarter_spec = lambda q: pl.BlockSpec(
        (_EBLK4, D_EDGE), lambda i, q=q: (q * _NBLK + i, 0))
    return pl.pallas_call(
        _edge_table_body,
        grid=(_NBLK,),
        in_specs=[
            quarter_spec(0), quarter_spec(1), quarter_spec(2), quarter_spec(3),
            pl.BlockSpec((D_EDGE, DP), lambda i: (0, 0)),
            pl.BlockSpec((1, DP), lambda i: (0, 0)),
        ],
        out_specs=pl.BlockSpec((_EBLK4, 4 * DP), lambda i: (i, 0)),
        out_shape=jax.ShapeDtypeStruct((E4, 4 * DP), jnp.float32),
    )(edge_attr, edge_attr, edge_attr, edge_attr, w3p, cbias)


def _edge_agg_body(idx_hbm, a_hbm, b_hbm, c_hbm, out_hbm,
                   dst_all, src_all, a0, a1, b0, b1, c0, c1, z_v, acc_sh,
                   sa0, sa1, sb0, sb1, sc0, sc1, ss0, ss1):
    cid = lax.axis_index("c")
    sid = lax.axis_index("s")
    wid = cid * NS + sid
    slots = ((a0, b0, c0, sa0, sb0, sc0, ss0),
             (a1, b1, c1, sa1, sb1, sc1, ss1))
    nchunk = jnp.where(wid == NW - 1, (N_EDGES - (NW - 1) * EPW) // CH, NCHUNK)

    pltpu.sync_copy(idx_hbm.at[wid], src_all)
    pltpu.sync_copy(idx_hbm.at[NW + wid], dst_all)

    def gathers(cn, slot):
        av, bv, cv, sa, sb, sc, _ = slot
        pltpu.async_copy(a_hbm.at[dst_all.at[cn]], av, sa)
        pltpu.async_copy(b_hbm.at[src_all.at[cn]], bv, sb)
        pltpu.async_copy(
            c_hbm.at[pl.ds(wid * (EPW * DP) + cn * (CH * DP), CH * DP)], cv, sc)

    def wait_gathers(cn, slot):
        av, bv, cv, sa, sb, sc, _ = slot
        pltpu.make_async_copy(a_hbm.at[dst_all.at[cn]], av, sa).wait()
        pltpu.make_async_copy(b_hbm.at[src_all.at[cn]], bv, sb).wait()
        pltpu.make_async_copy(
            c_hbm.at[pl.ds(wid * (EPW * DP) + cn * (CH * DP), CH * DP)],
            cv, sc).wait()

    def wait_scatter(cp, slot):
        av, ss = slot[0], slot[6]
        pltpu.make_async_copy(av, acc_sh.at[dst_all.at[cp]], ss).wait()

    gathers(0, slots[0])

    def zrow(j, carry):
        z_v[j, pl.ds(0, 16)] = jnp.zeros((16,), jnp.float32)
        z_v[j, pl.ds(16, 16)] = jnp.zeros((16,), jnp.float32)
        return carry

    lax.fori_loop(0, RPS, zrow, 0)
    pltpu.sync_copy(z_v, acc_sh.at[pl.ds(sid * RPS, RPS)])
    plsc.subcore_barrier()

    def pair(t, carry):
        for b in (0, 1):
            ci = 2 * t + b
            nxt = ci + 1
            cur, nsl = slots[b], slots[1 - b]

            @pl.when(nxt < nchunk)
            def _prefetch():
                @pl.when(ci >= 1)
                def _free():
                    wait_scatter(ci - 1, nsl)
                gathers(nxt, nsl)

            wait_gathers(ci, cur)
            av, bv, cv = cur[0], cur[1], cur[2]

            def mrow(r, inner):
                for q in range(4):
                    for k in range(DP // 16):
                        sl = pl.ds(k * 16, 16)
                        j = q * CH4 + r
                        m = (av[j, sl] + bv[j, sl]
                             + cv[pl.ds(r * 128 + q * DP + k * 16, 16)])
                        av[j, sl] = jnp.maximum(m, 0.0)
                return inner

            lax.fori_loop(0, CH4, mrow, 0)
            pltpu.async_copy(av, acc_sh.at[dst_all.at[ci]], cur[6], add=True)
        return carry

    lax.fori_loop(0, nchunk // 2, pair, 0)
    wait_scatter(0, slots[0])
    wait_scatter(0, slots[1])
    plsc.subcore_barrier()

    pltpu.sync_copy(acc_sh.at[pl.ds(sid * RPS, RPS)], z_v)
    pltpu.sync_copy(z_v, out_hbm.at[cid, pl.ds(sid * RPS, RPS)])


def _edge_agg(idx3, a, b, c_flat):
    mesh = plsc.VectorSubcoreMesh(
        core_axis_name="c", subcore_axis_name="s",
        num_cores=NC, num_subcores=NS,
    )
    f = functools.partial(
        pl.kernel,
        out_type=jax.ShapeDtypeStruct((NC, NP, DP), jnp.float32),
        mesh=mesh,
        scratch_types=[
            pltpu.VMEM((NCHUNK, CH), jnp.int32),
            pltpu.VMEM((NCHUNK, CH), jnp.int32),
            pltpu.VMEM((CH, DP), jnp.float32),
            pltpu.VMEM((CH, DP), jnp.float32),
            pltpu.VMEM((CH, DP), jnp.float32),
            pltpu.VMEM((CH, DP), jnp.float32),
            pltpu.VMEM((CH * DP,), jnp.float32),
            pltpu.VMEM((CH * DP,), jnp.float32),
            pltpu.VMEM((RPS, DP), jnp.float32),
            pltpu.VMEM_SHARED((NP, DP), jnp.float32),
            pltpu.SemaphoreType.DMA,
            pltpu.SemaphoreType.DMA,
            pltpu.SemaphoreType.DMA,
            pltpu.SemaphoreType.DMA,
            pltpu.SemaphoreType.DMA,
            pltpu.SemaphoreType.DMA,
            pltpu.SemaphoreType.DMA,
            pltpu.SemaphoreType.DMA,
        ],
        compiler_params=pltpu.CompilerParams(use_tc_tiling_on_sc=False),
    )(_edge_agg_body)
    return f(idx3, a, b, c_flat)


def kernel(x, edge_index, edge_attr, batch, W_emb, b_emb, W_conv, b_conv,
           W_post, b_post, W_f1, b_f1, W_f2, b_f2, W_f3, b_f3):
    W1 = W_conv[:, :D_NODE]
    W2 = W_conv[:, D_NODE:2 * D_NODE]
    W3 = W_conv[:, 2 * D_NODE:]
    w12T = jnp.zeros((D_NODE, 2 * DP), jnp.float32)
    w12T = w12T.at[:, :D_NODE].set(W1.T)
    w12T = w12T.at[:, DP:DP + D_NODE].set(W2.T)
    w3p = jnp.zeros((D_EDGE, DP), jnp.float32)
    w3p = w3p.at[:, :D_NODE].set(W3.T)
    cbias = jnp.zeros((DP,), jnp.float32)
    cbias = cbias.at[:D_NODE].set(b_conv)
    cbias = cbias.at[CNT].set(1.0)

    xp = jnp.pad(x, ((0, NP - N_NODES), (0, 0)))
    idxp = edge_index.reshape(2, 4, E4)
    idxp = jnp.pad(idxp, ((0, 0), (0, 0), (0, NW * EPW // 4 - E4)))
    idxp = idxp.reshape(2, 4, NW, NCHUNK, CH4).transpose(0, 2, 3, 1, 4)
    idx3 = idxp.reshape(2 * NW, NCHUNK, CH)

    a_tab, b_tab = _node_tables(xp, W_emb.T, b_emb[None, :], w12T)
    c_tab = _edge_table(edge_attr, w3p, cbias[None, :])
    parts = _edge_agg(idx3, a_tab, b_tab, c_tab.reshape(-1))

    s_acc = parts[0, :N_NODES] + parts[1, :N_NODES]
    cnt = jnp.clip(s_acc[:, CNT], 1.0)
    h2 = s_acc[:, :D_NODE] / cnt[:, None]
    h3 = jax.nn.softplus(h2 @ W_post.T + b_post)
    onehot = (batch[:, None] == jnp.arange(G_POOL)[None, :]).astype(jnp.float32)
    psum = onehot.T @ h3
    pcnt = jnp.clip(jnp.sum(onehot, axis=0), 1.0)
    pooled = psum / pcnt[:, None]
    out = jax.nn.relu(pooled @ W_f1.T + b_f1)
    out = jax.nn.relu(out @ W_f2.T + b_f2)
    return out @ W_f3.T + b_f3
